# Initial kernel scaffold; baseline (speedup 1.0000x reference)
import functools

import jax
import jax.numpy as jnp
from jax import lax
from jax.experimental import pallas as pl
from jax.experimental.pallas import tpu as pltpu
from jax.experimental.pallas import tpu_sc as plsc

D = 128
DP = 144
NEG_SLOPE = 0.2
EPS = 1e-16
NC, NS, NW = 2, 16, 32
K = 128
BLK = 1000


def _tc_head_body(is_first, in_ref, w_ref, asrc_ref, adst_ref,
                  hpad_ref, ssrc_ref, sdst_ref):
    if is_first:
        x = in_ref[...]
    else:
        a = in_ref[0] + in_ref[1]
        x = a[:, :D] / (a[:, D:D + 1] + EPS)
    w = w_ref[...]
    vsrc = jnp.dot(w, asrc_ref[0, :][:, None],
                   preferred_element_type=jnp.float32)
    vdst = jnp.dot(w, adst_ref[0, :][:, None],
                   preferred_element_type=jnp.float32)
    h = jnp.dot(x, w, preferred_element_type=jnp.float32)
    b = h.shape[0]
    pad = jnp.concatenate(
        [h, jnp.ones((b, 1), jnp.float32), jnp.zeros((b, DP - D - 1),
                                                     jnp.float32)], axis=1)
    hpad_ref[...] = pad
    ssrc_ref[...] = jnp.dot(x, vsrc, preferred_element_type=jnp.float32)
    sdst_ref[...] = jnp.dot(x, vdst, preferred_element_type=jnp.float32)


def _tc_head(x_or_agg, w, asrc, adst, n, is_first):
    grid = (n // BLK,)
    if is_first:
        in_spec = pl.BlockSpec((BLK, D), lambda i: (i, 0))
    else:
        in_spec = pl.BlockSpec((NC, BLK, DP), lambda i: (0, i, 0))
    return pl.pallas_call(
        functools.partial(_tc_head_body, is_first),
        grid=grid,
        in_specs=[
            in_spec,
            pl.BlockSpec((D, D), lambda i: (0, 0)),
            pl.BlockSpec((1, D), lambda i: (0, 0)),
            pl.BlockSpec((1, D), lambda i: (0, 0)),
        ],
        out_specs=[
            pl.BlockSpec((BLK, DP), lambda i: (i, 0)),
            pl.BlockSpec((BLK, 1), lambda i: (i, 0)),
            pl.BlockSpec((BLK, 1), lambda i: (i, 0)),
        ],
        out_shape=[
            jax.ShapeDtypeStruct((n, DP), jnp.float32),
            jax.ShapeDtypeStruct((n, 1), jnp.float32),
            jax.ShapeDtypeStruct((n, 1), jnp.float32),
        ],
    )(x_or_agg, w, asrc, adst)


def _tc_final_body(agg_ref, out_ref):
    a = agg_ref[0] + agg_ref[1]
    x = a[:, :D] / (a[:, D:D + 1] + EPS)
    out_ref[...] = jnp.where(x > 0, x, jnp.expm1(x))


def _tc_final(agg, n):
    return pl.pallas_call(
        _tc_final_body,
        grid=(n // BLK,),
        in_specs=[pl.BlockSpec((NC, BLK, DP), lambda i: (0, i, 0))],
        out_specs=pl.BlockSpec((BLK, D), lambda i: (i, 0)),
        out_shape=jax.ShapeDtypeStruct((n, D), jnp.float32),
    )(agg)


def _make_sc_edge_pass(n, e):
    nchunk = e // K
    per, rem = nchunk // NW, nchunk % NW
    rows_per_sub = n // NS
    dump = 125
    ndump = rows_per_sub // dump
    mesh = plsc.VectorSubcoreMesh(core_axis_name="c", subcore_axis_name="s")

    @functools.partial(
        pl.kernel,
        out_type=jax.ShapeDtypeStruct((NC, n, DP), jnp.float32),
        mesh=mesh,
        scratch_types=[
            pltpu.VMEM((K,), jnp.int32),
            pltpu.VMEM((K,), jnp.int32),
            pltpu.VMEM((K, DP), jnp.float32),
            pltpu.VMEM((K,), jnp.float32),
            pltpu.VMEM((n,), jnp.float32),
            pltpu.VMEM((n,), jnp.float32),
            pltpu.VMEM_SHARED((n, DP), jnp.float32),
            pltpu.SemaphoreType.DMA,
        ],
    )
    def sc_edge_pass(hpad_hbm, ssrc_hbm, sdst_hbm, src_hbm, dst_hbm,
                     out_hbm, idx_src, idx_dst, rows, exb, ssl, sdl,
                     agg, sem):
        c = lax.axis_index("c")
        s = lax.axis_index("s")
        w = s * NC + c

        def _zero_rows(k, _):
            for q in range(DP // 16):
                rows[k, pl.ds(q * 16, 16)] = jnp.zeros((16,), jnp.float32)
            return 0
        lax.fori_loop(0, K, _zero_rows, 0)
        for i in range(ndump):
            pltpu.sync_copy(rows.at[pl.ds(0, dump)],
                            agg.at[pl.ds(s * rows_per_sub + i * dump, dump)])

        pltpu.sync_copy(ssrc_hbm, ssl)
        pltpu.sync_copy(sdst_hbm, sdl)
        plsc.subcore_barrier()

        nj = per + jnp.where(w < rem, 1, 0)

        def chunk_body(j, _):
            base = (w + NW * j) * K
            pltpu.sync_copy(src_hbm.at[pl.ds(base, K)], idx_src)
            pltpu.sync_copy(dst_hbm.at[pl.ds(base, K)], idx_dst)
            cp = pltpu.async_copy(hpad_hbm.at[idx_src], rows, sem)
            for g in range(K // 16):
                sv = idx_src[pl.ds(g * 16, 16)]
                dv = idx_dst[pl.ds(g * 16, 16)]
                t = plsc.load_gather(ssl, [sv]) + plsc.load_gather(sdl, [dv])
                t = jnp.maximum(t, NEG_SLOPE * t)
                exb[pl.ds(g * 16, 16)] = jnp.exp(t)
            cp.wait()

            def mul_body(k, _):
                sc = exb[k]
                for q in range(DP // 16):
                    rows[k, pl.ds(q * 16, 16)] = rows[k, pl.ds(q * 16, 16)] * sc
                return 0
            lax.fori_loop(0, K, mul_body, 0)
            pltpu.sync_copy(rows, agg.at[idx_dst], add=True)
            return 0
        lax.fori_loop(0, nj, chunk_body, 0)

        plsc.subcore_barrier()
        for i in range(ndump):
            off = s * rows_per_sub + i * dump
            pltpu.sync_copy(agg.at[pl.ds(off, dump)],
                            out_hbm.at[c, pl.ds(off, dump)])

    return sc_edge_pass


def kernel(x, edge_index, W, a_src, a_dst):
    n = x.shape[0]
    e = edge_index.shape[1]
    src = edge_index[0]
    dst = edge_index[1]
    sc_pass = _make_sc_edge_pass(n, e)

    carry = x
    for i in range(W.shape[0]):
        hpad, ssrc, sdst = _tc_head(
            carry, W[i], a_src[i][None, :], a_dst[i][None, :], n,
            is_first=(i == 0))
        carry = sc_pass(hpad, ssrc.reshape(-1), sdst.reshape(-1), src, dst)
    return _tc_final(carry, n)


# speedup vs baseline: 26.2108x; 26.2108x over previous
import functools

import jax
import jax.numpy as jnp
from jax import lax
from jax.experimental import pallas as pl
from jax.experimental.pallas import tpu as pltpu
from jax.experimental.pallas import tpu_sc as plsc

D = 128
DP = 144
NEG_SLOPE = 0.2
EPS = 1e-16
NC, NS, NW = 2, 16, 32
K = 128
BLK = 1000


def _tc_head_body(is_first, in_ref, w_ref, asrc_ref, adst_ref,
                  hpad_ref, ssrc_ref, sdst_ref):
    if is_first:
        x = in_ref[...]
    else:
        a = in_ref[0] + in_ref[1]
        x = a[:, :D] / (a[:, D:D + 1] + EPS)
    w = w_ref[...]
    vsrc = jnp.dot(w, asrc_ref[0, :][:, None],
                   preferred_element_type=jnp.float32)
    vdst = jnp.dot(w, adst_ref[0, :][:, None],
                   preferred_element_type=jnp.float32)
    h = jnp.dot(x, w, preferred_element_type=jnp.float32)
    b = h.shape[0]
    pad = jnp.concatenate(
        [h, jnp.ones((b, 1), jnp.float32), jnp.zeros((b, DP - D - 1),
                                                     jnp.float32)], axis=1)
    hpad_ref[...] = pad
    ssrc_ref[...] = jnp.dot(x, vsrc, preferred_element_type=jnp.float32)
    sdst_ref[...] = jnp.dot(x, vdst, preferred_element_type=jnp.float32)


def _tc_head(x_or_agg, w, asrc, adst, n, is_first):
    grid = (n // BLK,)
    if is_first:
        in_spec = pl.BlockSpec((BLK, D), lambda i: (i, 0))
    else:
        in_spec = pl.BlockSpec((NC, BLK, DP), lambda i: (0, i, 0))
    return pl.pallas_call(
        functools.partial(_tc_head_body, is_first),
        grid=grid,
        in_specs=[
            in_spec,
            pl.BlockSpec((D, D), lambda i: (0, 0)),
            pl.BlockSpec((1, D), lambda i: (0, 0)),
            pl.BlockSpec((1, D), lambda i: (0, 0)),
        ],
        out_specs=[
            pl.BlockSpec((BLK, DP), lambda i: (i, 0)),
            pl.BlockSpec((BLK, 1), lambda i: (i, 0)),
            pl.BlockSpec((BLK, 1), lambda i: (i, 0)),
        ],
        out_shape=[
            jax.ShapeDtypeStruct((n, DP), jnp.float32),
            jax.ShapeDtypeStruct((n, 1), jnp.float32),
            jax.ShapeDtypeStruct((n, 1), jnp.float32),
        ],
    )(x_or_agg, w, asrc, adst)


def _tc_final_body(agg_ref, out_ref):
    a = agg_ref[0] + agg_ref[1]
    x = a[:, :D] / (a[:, D:D + 1] + EPS)
    out_ref[...] = jnp.where(x > 0, x, jnp.exp(jnp.minimum(x, 0.0)) - 1.0)


def _tc_final(agg, n):
    return pl.pallas_call(
        _tc_final_body,
        grid=(n // BLK,),
        in_specs=[pl.BlockSpec((NC, BLK, DP), lambda i: (0, i, 0))],
        out_specs=pl.BlockSpec((BLK, D), lambda i: (i, 0)),
        out_shape=jax.ShapeDtypeStruct((n, D), jnp.float32),
    )(agg)


def _make_sc_edge_pass(n, e):
    nchunk = e // K
    per, rem = nchunk // NW, nchunk % NW
    blk = 80
    nblk = n // blk
    bper, brem = nblk // NS, nblk % NS
    mesh = plsc.VectorSubcoreMesh(core_axis_name="c", subcore_axis_name="s")

    @functools.partial(
        pl.kernel,
        out_type=jax.ShapeDtypeStruct((NC, n, DP), jnp.float32),
        mesh=mesh,
        scratch_types=[
            pltpu.VMEM((K,), jnp.int32),
            pltpu.VMEM((K,), jnp.int32),
            pltpu.VMEM((K, DP), jnp.float32),
            pltpu.VMEM((K,), jnp.float32),
            pltpu.VMEM((n,), jnp.float32),
            pltpu.VMEM((n,), jnp.float32),
            pltpu.VMEM_SHARED((n, DP), jnp.float32),
            pltpu.SemaphoreType.DMA,
        ],
        compiler_params=pltpu.CompilerParams(needs_layout_passes=False,
                                             use_tc_tiling_on_sc=False),
    )
    def sc_edge_pass(hpad_hbm, ssrc_hbm, sdst_hbm, src_hbm, dst_hbm,
                     out_hbm, idx_src, idx_dst, rows, exb, ssl, sdl,
                     agg, sem):
        c = lax.axis_index("c")
        s = lax.axis_index("s")
        w = s * NC + c

        def _zero_rows(k, _):
            for q in range(DP // 16):
                rows[k, pl.ds(q * 16, 16)] = jnp.zeros((16,), jnp.float32)
            return 0
        lax.fori_loop(0, blk, _zero_rows, 0)
        nb = bper + jnp.where(s < brem, 1, 0)

        def _zero_blk(i, _):
            off = (s + NS * i) * blk
            pltpu.sync_copy(rows.at[pl.ds(0, blk)], agg.at[pl.ds(off, blk)])
            return 0
        lax.fori_loop(0, nb, _zero_blk, 0)

        pltpu.sync_copy(ssrc_hbm, ssl)
        pltpu.sync_copy(sdst_hbm, sdl)
        plsc.subcore_barrier()

        nj = per + jnp.where(w < rem, 1, 0)

        def chunk_body(j, _):
            base = (w + NW * j) * K
            pltpu.sync_copy(src_hbm.at[pl.ds(base, K)], idx_src)
            pltpu.sync_copy(dst_hbm.at[pl.ds(base, K)], idx_dst)
            cp = pltpu.async_copy(hpad_hbm.at[idx_src], rows, sem)
            for g in range(K // 16):
                sv = idx_src[pl.ds(g * 16, 16)]
                dv = idx_dst[pl.ds(g * 16, 16)]
                t = plsc.load_gather(ssl, [sv]) + plsc.load_gather(sdl, [dv])
                t = jnp.maximum(t, NEG_SLOPE * t)
                exb[pl.ds(g * 16, 16)] = jnp.exp(t)
            cp.wait()

            def mul_body(g, _):
                ex_v = exb[pl.ds(g * 16, 16)]
                for j in range(16):
                    sc = ex_v[j]
                    k = g * 16 + j
                    for q in range(DP // 16):
                        rows[k, pl.ds(q * 16, 16)] = (
                            rows[k, pl.ds(q * 16, 16)] * sc)
                return 0
            lax.fori_loop(0, K // 16, mul_body, 0)
            pltpu.sync_copy(rows, agg.at[idx_dst], add=True)
            return 0
        lax.fori_loop(0, nj, chunk_body, 0)

        plsc.subcore_barrier()

        def _dump_blk(i, _):
            off = (s + NS * i) * blk
            pltpu.sync_copy(agg.at[pl.ds(off, blk)],
                            out_hbm.at[c, pl.ds(off, blk)])
            return 0
        lax.fori_loop(0, nb, _dump_blk, 0)

    return sc_edge_pass


def kernel(x, edge_index, W, a_src, a_dst):
    n = x.shape[0]
    e = edge_index.shape[1]
    src = edge_index[0]
    dst = edge_index[1]
    sc_pass = _make_sc_edge_pass(n, e)

    carry = x
    for i in range(W.shape[0]):
        hpad, ssrc, sdst = _tc_head(
            carry, W[i], a_src[i][None, :], a_dst[i][None, :], n,
            is_first=(i == 0))
        carry = sc_pass(hpad, ssrc.reshape(-1), sdst.reshape(-1), src, dst)
    return _tc_final(carry, n)
